# Initial kernel scaffold; baseline (speedup 1.0000x reference)
import jax
import jax.numpy as jnp
from jax.experimental import pallas as pl


def kernel(x, gate_w, w1, w2):
    raise NotImplementedError("write your pallas kernel here")


# speedup vs baseline: 1.0803x; 1.0803x over previous
import jax
import jax.numpy as jnp
from jax.experimental import pallas as pl
from jax.experimental.pallas import tpu as pltpu

E = 64
TOPK = 8
H = 1024
I = 768


def _moe_body(x_ref, gw_ref, w1_ref, w2_ref, out_ref, comb_ref):
    e = pl.program_id(0)
    T = x_ref.shape[0]
    xt = x_ref[...]

    @pl.when(e == 0)
    def _router():
        logits = jax.lax.dot_general(
            xt, gw_ref[...], (((1,), (1,)), ((), ())),
            preferred_element_type=jnp.float32)
        m = jnp.max(logits, axis=-1, keepdims=True)
        ex = jnp.exp(logits - m)
        probs = ex / jnp.sum(ex, axis=-1, keepdims=True)
        col = jax.lax.broadcasted_iota(jnp.int32, (T, E), 1)
        work = probs
        comb = jnp.zeros_like(probs)
        ssum = jnp.zeros((T, 1), jnp.float32)
        for _ in range(TOPK):
            mx = jnp.max(work, axis=-1, keepdims=True)
            sel = jnp.where(work == mx, col, E)
            j = jnp.min(sel, axis=-1, keepdims=True)
            chosen = col == j
            comb = comb + jnp.where(chosen, probs, 0.0)
            ssum = ssum + mx
            work = jnp.where(chosen, -jnp.inf, work)
        comb_ref[...] = comb / ssum

    h = jax.lax.dot_general(
        xt, w1_ref[0], (((1,), (1,)), ((), ())),
        preferred_element_type=jnp.float32)
    a = h * jax.nn.sigmoid(h)
    ecol = jax.lax.broadcasted_iota(jnp.int32, (T, E), 1)
    c = jnp.sum(jnp.where(ecol == e, comb_ref[...], 0.0), axis=-1,
                keepdims=True)
    a = a * c
    y = jax.lax.dot_general(
        a, w2_ref[0], (((1,), (1,)), ((), ())),
        preferred_element_type=jnp.float32)

    @pl.when(e == 0)
    def _init():
        out_ref[...] = y

    @pl.when(e > 0)
    def _acc():
        out_ref[...] += y


def kernel(x, gate_w, w1, w2):
    orig_shape = x.shape
    xt = x.reshape(-1, x.shape[-1])
    T = xt.shape[0]
    out = pl.pallas_call(
        _moe_body,
        grid=(E,),
        in_specs=[
            pl.BlockSpec((T, H), lambda e: (0, 0)),
            pl.BlockSpec((E, H), lambda e: (0, 0)),
            pl.BlockSpec((1, I, H), lambda e: (e, 0, 0)),
            pl.BlockSpec((1, H, I), lambda e: (e, 0, 0)),
        ],
        out_specs=pl.BlockSpec((T, H), lambda e: (0, 0)),
        out_shape=jax.ShapeDtypeStruct((T, H), jnp.float32),
        scratch_shapes=[pltpu.VMEM((T, E), jnp.float32)],
        compiler_params=pltpu.CompilerParams(
            dimension_semantics=("arbitrary",)),
    )(xt, gate_w, w1, w2)
    return out.reshape(orig_shape)
